# baseline (speedup 1.0000x reference)
import jax
import jax.numpy as jnp
from jax import lax
from jax.experimental import pallas as pl
from jax.experimental.pallas import tpu as pltpu
from jax.experimental.pallas import tpu_sc as plsc

_K = 99
_PAD_K = 128
_STRIDE = 129
_EPS = 1e-05
_D = 26
_DP = 32
_B = 16384
_NW = 32
_ROWS = _B // _NW


def _body(x_hbm, q_hbm, p_hbm, m_hbm, s_hbm, out_hbm,
          x_v, o_v, q_v, p_v, m_v, s_v, qpad_v, qlpad_v, ppad_v, spad_v, sem):
    wid = lax.axis_index("s") * 2 + lax.axis_index("c")
    rbase = wid * _ROWS

    xcopy = pltpu.async_copy(x_hbm.at[pl.ds(rbase, _ROWS), :], x_v, sem)
    pltpu.sync_copy(q_hbm, q_v)
    pltpu.sync_copy(p_hbm, p_v)
    pltpu.sync_copy(m_hbm, m_v)
    pltpu.sync_copy(s_hbm, s_v)

    @plsc.parallel_loop(0, _D * _PAD_K // 16, step=1, unroll=4)
    def build(j):
        flat = j * 16 + lax.iota(jnp.int32, 16)
        d = lax.shift_right_logical(flat, 7)
        c = lax.bitwise_and(flat, _PAD_K - 1)
        cc = jnp.minimum(c, _K - 1)
        cm = jnp.minimum(jnp.maximum(c - 1, 0), _K - 1)
        addr = flat + d
        sv = plsc.load_gather(s_v, [d])
        mv = plsc.load_gather(m_v, [d])
        qsk = plsc.load_gather(q_v, [d, cc]) * sv + mv
        qsm = plsc.load_gather(q_v, [d, cm]) * sv + mv
        pk = plsc.load_gather(p_v, [cc])
        pm = plsc.load_gather(p_v, [cm])
        slope = (pk - pm) / (qsk - qsm + _EPS * sv)
        slope = jnp.where((c == 0) | (c > _K - 1), 0.0, slope)
        plsc.store_scatter(qpad_v, [addr],
                           jnp.where(c > _K - 1, jnp.inf, qsk))
        plsc.store_scatter(qlpad_v, [addr], qsm)
        plsc.store_scatter(ppad_v, [addr], pm)
        plsc.store_scatter(spad_v, [addr], slope)

    iota16 = lax.iota(jnp.int32, 16)
    d0 = iota16
    d1 = (iota16 + 16) % _D
    sv0 = plsc.load_gather(s_v, [d0])
    sv1 = plsc.load_gather(s_v, [d1])
    halves = []
    for col, dd, sv in ((0, d0, sv0), (16, d1, sv1)):
        qb = dd * _STRIDE
        q63 = plsc.load_gather(qpad_v, [qb + 63])
        q31 = plsc.load_gather(qpad_v, [qb + 31])
        q95 = plsc.load_gather(qpad_v, [qb + 95])
        halves.append((col, qb, 1.0 / sv, q63, q31, q95))

    xcopy.wait()

    @plsc.parallel_loop(0, _ROWS, step=1, unroll=8)
    def body(r):
        for (col, qb, iv, q63, q31, q95) in halves:
            xv = x_v[r, pl.ds(col, 16)]

            apos = jnp.where(q63 < xv, qb + 64, qb)
            apos = jnp.where(jnp.where(q63 < xv, q95, q31) < xv,
                             apos + 32, apos)
            for step in (16, 8, 4, 2, 1):
                qv = plsc.load_gather(qpad_v, [apos + (step - 1)])
                apos = jnp.where(qv < xv, apos + step, apos)
            idx = apos - qb

            qls = plsc.load_gather(qlpad_v, [apos])
            pL = plsc.load_gather(ppad_v, [apos])
            sl = plsc.load_gather(spad_v, [apos])

            res = pL + (xv - qls) * sl
            mlow = (idx == 0) & (xv < qls)
            mhigh = (idx == _K) & (xv > qls)
            ta = jnp.abs(xv - qls) * iv
            e = jnp.exp(-2.0 * ta)
            th = (1.0 - e) / (1.0 + e)
            res = jnp.where(mlow, pL - pL * th, res)
            res = jnp.where(mhigh, pL + (1.0 - pL) * th, res)
            o_v[r, pl.ds(col, 16)] = res

    pltpu.sync_copy(o_v, out_hbm.at[pl.ds(rbase, _ROWS), :])


@jax.jit
def _qnorm(xp, quantiles, probs, initial_means, initial_stds):
    mesh = plsc.VectorSubcoreMesh(core_axis_name="c", subcore_axis_name="s")
    f = pl.kernel(
        _body,
        out_type=jax.ShapeDtypeStruct((_B, _DP), jnp.float32),
        mesh=mesh,
        compiler_params=pltpu.CompilerParams(
            needs_layout_passes=False, use_tc_tiling_on_sc=False),
        scratch_types=[
            pltpu.VMEM((_ROWS, _DP), jnp.float32),
            pltpu.VMEM((_ROWS, _DP), jnp.float32),
            pltpu.VMEM((_D, _K), jnp.float32),
            pltpu.VMEM((_K,), jnp.float32),
            pltpu.VMEM((_D,), jnp.float32),
            pltpu.VMEM((_D,), jnp.float32),
            pltpu.VMEM((_D * _STRIDE,), jnp.float32),
            pltpu.VMEM((_D * _STRIDE,), jnp.float32),
            pltpu.VMEM((_D * _STRIDE,), jnp.float32),
            pltpu.VMEM((_D * _STRIDE,), jnp.float32),
            pltpu.SemaphoreType.DMA,
        ],
    )
    return f(xp, quantiles, probs, initial_means, initial_stds)


def kernel(x, quantiles, probs, initial_means, initial_stds):
    xp = jnp.pad(x, ((0, 0), (0, _DP - _D)))
    out = _qnorm(xp, quantiles, probs, initial_means, initial_stds)
    return out[:, :_D]


# speedup vs baseline: 1.0829x; 1.0829x over previous
import jax
import jax.numpy as jnp
from jax import lax
from jax.experimental import pallas as pl
from jax.experimental.pallas import tpu as pltpu
from jax.experimental.pallas import tpu_sc as plsc

_K = 99
_PAD_K = 128
_STRIDE = 129
_EPS = 1e-05
_D = 26
_DP = 32
_B = 16384
_NW = 32
_ROWS = _B // _NW


def _body(x_hbm, q_hbm, p_hbm, m_hbm, s_hbm, out_hbm,
          x_v, o_v, q_v, p_v, m_v, s_v, qpad_v, qlpad_v, ppad_v, spad_v, sem):
    wid = lax.axis_index("s") * 2 + lax.axis_index("c")
    rbase = wid * _ROWS

    xcopy = pltpu.async_copy(x_hbm.at[pl.ds(rbase, _ROWS), :], x_v, sem)
    pltpu.sync_copy(q_hbm, q_v)
    pltpu.sync_copy(p_hbm, p_v)
    pltpu.sync_copy(m_hbm, m_v)
    pltpu.sync_copy(s_hbm, s_v)

    @plsc.parallel_loop(0, _D * _PAD_K // 16, step=1, unroll=4)
    def build(j):
        flat = j * 16 + lax.iota(jnp.int32, 16)
        d = lax.shift_right_logical(flat, 7)
        c = lax.bitwise_and(flat, _PAD_K - 1)
        cc = jnp.minimum(c, _K - 1)
        cm = jnp.minimum(jnp.maximum(c - 1, 0), _K - 1)
        addr = flat + d
        sv = plsc.load_gather(s_v, [d])
        mv = plsc.load_gather(m_v, [d])
        qsk = plsc.load_gather(q_v, [d, cc]) * sv + mv
        qsm = plsc.load_gather(q_v, [d, cm]) * sv + mv
        pk = plsc.load_gather(p_v, [cc])
        pm = plsc.load_gather(p_v, [cm])
        slope = (pk - pm) / (qsk - qsm + _EPS * sv)
        slope = jnp.where((c == 0) | (c > _K - 1), 0.0, slope)
        plsc.store_scatter(qpad_v, [addr],
                           jnp.where(c > _K - 1, jnp.inf, qsk))
        plsc.store_scatter(qlpad_v, [addr], qsm)
        plsc.store_scatter(ppad_v, [addr], pm)
        plsc.store_scatter(spad_v, [addr], slope)

    iota16 = lax.iota(jnp.int32, 16)
    d0 = iota16
    d1 = (iota16 + 16) % _D
    sv0 = plsc.load_gather(s_v, [d0])
    sv1 = plsc.load_gather(s_v, [d1])
    halves = []
    for col, dd, sv in ((0, d0, sv0), (16, d1, sv1)):
        qb = dd * _STRIDE
        q63 = plsc.load_gather(qpad_v, [qb + 63])
        q31 = plsc.load_gather(qpad_v, [qb + 31])
        q95 = plsc.load_gather(qpad_v, [qb + 95])
        halves.append((col, qb, 1.0 / sv, q63, q31, q95))

    xcopy.wait()

    @plsc.parallel_loop(0, _ROWS, step=1, unroll=4)
    def body(r):
        for (col, qb, iv, q63, q31, q95) in halves:
            xv = x_v[r, pl.ds(col, 16)]

            apos = jnp.where(q63 < xv, qb + 64, qb)
            apos = jnp.where(jnp.where(q63 < xv, q95, q31) < xv,
                             apos + 32, apos)
            for step in (16, 8, 4, 2, 1):
                qv = plsc.load_gather(qpad_v, [apos + (step - 1)])
                apos = jnp.where(qv < xv, apos + step, apos)
            idx = apos - qb

            qls = plsc.load_gather(qlpad_v, [apos])
            pL = plsc.load_gather(ppad_v, [apos])
            sl = plsc.load_gather(spad_v, [apos])

            res = pL + (xv - qls) * sl
            mlow = (idx == 0) & (xv < qls)
            mhigh = (idx == _K) & (xv > qls)
            ta = jnp.abs(xv - qls) * iv
            e = jnp.exp(-2.0 * ta)
            th = (1.0 - e) / (1.0 + e)
            res = jnp.where(mlow, pL - pL * th, res)
            res = jnp.where(mhigh, pL + (1.0 - pL) * th, res)
            o_v[r, pl.ds(col, 16)] = res

    pltpu.sync_copy(o_v, out_hbm.at[pl.ds(rbase, _ROWS), :])


@jax.jit
def _qnorm(xp, quantiles, probs, initial_means, initial_stds):
    mesh = plsc.VectorSubcoreMesh(core_axis_name="c", subcore_axis_name="s")
    f = pl.kernel(
        _body,
        out_type=jax.ShapeDtypeStruct((_B, _DP), jnp.float32),
        mesh=mesh,
        compiler_params=pltpu.CompilerParams(
            needs_layout_passes=False, use_tc_tiling_on_sc=False),
        scratch_types=[
            pltpu.VMEM((_ROWS, _DP), jnp.float32),
            pltpu.VMEM((_ROWS, _DP), jnp.float32),
            pltpu.VMEM((_D, _K), jnp.float32),
            pltpu.VMEM((_K,), jnp.float32),
            pltpu.VMEM((_D,), jnp.float32),
            pltpu.VMEM((_D,), jnp.float32),
            pltpu.VMEM((_D * _STRIDE,), jnp.float32),
            pltpu.VMEM((_D * _STRIDE,), jnp.float32),
            pltpu.VMEM((_D * _STRIDE,), jnp.float32),
            pltpu.VMEM((_D * _STRIDE,), jnp.float32),
            pltpu.SemaphoreType.DMA,
        ],
    )
    return f(xp, quantiles, probs, initial_means, initial_stds)


def kernel(x, quantiles, probs, initial_means, initial_stds):
    xp = jnp.pad(x, ((0, 0), (0, _DP - _D)))
    out = _qnorm(xp, quantiles, probs, initial_means, initial_stds)
    return out[:, :_D]


# speedup vs baseline: 1.2569x; 1.1606x over previous
import jax
import jax.numpy as jnp
from jax import lax
from jax.experimental import pallas as pl
from jax.experimental.pallas import tpu as pltpu
from jax.experimental.pallas import tpu_sc as plsc

_K = 99
_PAD_K = 128
_STRIDE = 129
_EPS = 1e-05
_D = 26
_DP = 32
_B = 16384
_NW = 32
_ROWS = _B // _NW
_CHUNKS = 2
_CROWS = _ROWS // _CHUNKS


def _body(x_hbm, q_hbm, p_hbm, m_hbm, s_hbm, out_hbm,
          x_v, o_v, q_v, p_v, m_v, s_v, qpad_v, qlpad_v, ppad_v, spad_v, sem):
    wid = lax.axis_index("s") * 2 + lax.axis_index("c")
    rbase = wid * _ROWS

    xcopy = pltpu.async_copy(x_hbm.at[pl.ds(rbase, _CROWS), :], x_v, sem)
    pltpu.sync_copy(q_hbm, q_v)
    pltpu.sync_copy(p_hbm, p_v)
    pltpu.sync_copy(m_hbm, m_v)
    pltpu.sync_copy(s_hbm, s_v)

    @plsc.parallel_loop(0, _D * _PAD_K // 16, step=1, unroll=4)
    def build(j):
        flat = j * 16 + lax.iota(jnp.int32, 16)
        d = lax.shift_right_logical(flat, 7)
        c = lax.bitwise_and(flat, _PAD_K - 1)
        cc = jnp.minimum(c, _K - 1)
        cm = jnp.minimum(jnp.maximum(c - 1, 0), _K - 1)
        addr = flat + d
        sv = plsc.load_gather(s_v, [d])
        mv = plsc.load_gather(m_v, [d])
        qsk = plsc.load_gather(q_v, [d, cc]) * sv + mv
        qsm = plsc.load_gather(q_v, [d, cm]) * sv + mv
        pk = plsc.load_gather(p_v, [cc])
        pm = plsc.load_gather(p_v, [cm])
        slope = (pk - pm) / (qsk - qsm + _EPS * sv)
        slope = jnp.where((c == 0) | (c > _K - 1), 0.0, slope)
        plsc.store_scatter(qpad_v, [addr],
                           jnp.where(c > _K - 1, jnp.inf, qsk))
        plsc.store_scatter(qlpad_v, [addr], qsm)
        plsc.store_scatter(ppad_v, [addr], pm)
        plsc.store_scatter(spad_v, [addr], slope)

    iota16 = lax.iota(jnp.int32, 16)
    d0 = iota16
    d1 = (iota16 + 16) % _D
    sv0 = plsc.load_gather(s_v, [d0])
    sv1 = plsc.load_gather(s_v, [d1])
    halves = []
    for col, dd, sv in ((0, d0, sv0), (16, d1, sv1)):
        qb = dd * _STRIDE
        q63 = plsc.load_gather(qpad_v, [qb + 63])
        q31 = plsc.load_gather(qpad_v, [qb + 31])
        q95 = plsc.load_gather(qpad_v, [qb + 95])
        halves.append((col, qb, 1.0 / sv, q63, q31, q95))

    xcopy.wait()

    for chunk in range(_CHUNKS):
        if chunk > 0:
            pltpu.sync_copy(
                x_hbm.at[pl.ds(rbase + chunk * _CROWS, _CROWS), :], x_v)

        @plsc.parallel_loop(0, _CROWS, step=1, unroll=4)
        def body(r):
            for (col, qb, iv, q63, q31, q95) in halves:
                xv = x_v[r, pl.ds(col, 16)]

                apos = jnp.where(q63 < xv, qb + 64, qb)
                apos = jnp.where(jnp.where(q63 < xv, q95, q31) < xv,
                                 apos + 32, apos)
                for step in (16, 8, 4, 2, 1):
                    qv = plsc.load_gather(qpad_v, [apos + (step - 1)])
                    apos = jnp.where(qv < xv, apos + step, apos)
                idx = apos - qb

                qls = plsc.load_gather(qlpad_v, [apos])
                pL = plsc.load_gather(ppad_v, [apos])
                sl = plsc.load_gather(spad_v, [apos])

                res = pL + (xv - qls) * sl
                mlow = (idx == 0) & (xv < qls)
                mhigh = (idx == _K) & (xv > qls)
                ta = jnp.abs(xv - qls) * iv
                e = jnp.exp(-2.0 * ta)
                th = (1.0 - e) / (1.0 + e)
                res = jnp.where(mlow, pL - pL * th, res)
                res = jnp.where(mhigh, pL + (1.0 - pL) * th, res)
                o_v[r, pl.ds(col, 16)] = res

        pltpu.sync_copy(
            o_v, out_hbm.at[pl.ds(rbase + chunk * _CROWS, _CROWS), :])


@jax.jit
def _qnorm(xp, quantiles, probs, initial_means, initial_stds):
    mesh = plsc.VectorSubcoreMesh(core_axis_name="c", subcore_axis_name="s")
    f = pl.kernel(
        _body,
        out_type=jax.ShapeDtypeStruct((_B, _DP), jnp.float32),
        mesh=mesh,
        compiler_params=pltpu.CompilerParams(
            needs_layout_passes=False, use_tc_tiling_on_sc=True),
        scratch_types=[
            pltpu.VMEM((_CROWS, _DP), jnp.float32),
            pltpu.VMEM((_CROWS, _DP), jnp.float32),
            pltpu.VMEM((_D, _K), jnp.float32),
            pltpu.VMEM((_K,), jnp.float32),
            pltpu.VMEM((_D,), jnp.float32),
            pltpu.VMEM((_D,), jnp.float32),
            pltpu.VMEM((_D * _STRIDE,), jnp.float32),
            pltpu.VMEM((_D * _STRIDE,), jnp.float32),
            pltpu.VMEM((_D * _STRIDE,), jnp.float32),
            pltpu.VMEM((_D * _STRIDE,), jnp.float32),
            pltpu.SemaphoreType.DMA,
        ],
    )
    return f(xp, quantiles, probs, initial_means, initial_stds)


def kernel(x, quantiles, probs, initial_means, initial_stds):
    xp = jnp.pad(x, ((0, 0), (0, _DP - _D)))
    out = _qnorm(xp, quantiles, probs, initial_means, initial_stds)
    return out[:, :_D]


# speedup vs baseline: 1.3645x; 1.0856x over previous
import jax
import jax.numpy as jnp
from jax import lax
from jax.experimental import pallas as pl
from jax.experimental.pallas import tpu as pltpu
from jax.experimental.pallas import tpu_sc as plsc

_K = 99
_PAD_K = 128
_STRIDE = 129
_EPS = 1e-05
_D = 26
_DP = 32
_B = 16384
_NW = 32
_ROWS = _B // _NW
_CHUNKS = 2
_CROWS = _ROWS // _CHUNKS


def _body(x_hbm, q_hbm, p_hbm, m_hbm, s_hbm, out_hbm,
          x_v, o_v, q_v, p_v, m_v, s_v, qpad_v, qlpad_v, ppad_v, spad_v, sem):
    wid = lax.axis_index("s") * 2 + lax.axis_index("c")
    rbase = wid * _ROWS

    xcopy = pltpu.async_copy(x_hbm.at[pl.ds(rbase, _CROWS), :], x_v, sem)
    pltpu.sync_copy(q_hbm, q_v)
    pltpu.sync_copy(p_hbm, p_v)
    pltpu.sync_copy(m_hbm, m_v)
    pltpu.sync_copy(s_hbm, s_v)

    @plsc.parallel_loop(0, _D * _PAD_K // 16, step=1, unroll=4)
    def build(j):
        flat = j * 16 + lax.iota(jnp.int32, 16)
        d = lax.shift_right_logical(flat, 7)
        c = lax.bitwise_and(flat, _PAD_K - 1)
        cc = jnp.minimum(c, _K - 1)
        cm = jnp.minimum(jnp.maximum(c - 1, 0), _K - 1)
        addr = flat + d
        sv = plsc.load_gather(s_v, [d])
        mv = plsc.load_gather(m_v, [d])
        qsk = plsc.load_gather(q_v, [d, cc]) * sv + mv
        qsm = plsc.load_gather(q_v, [d, cm]) * sv + mv
        pk = plsc.load_gather(p_v, [cc])
        pm = plsc.load_gather(p_v, [cm])
        slope = (pk - pm) / (qsk - qsm + _EPS * sv)
        slope = jnp.where((c == 0) | (c > _K - 1), 0.0, slope)
        plsc.store_scatter(qpad_v, [addr],
                           jnp.where(c > _K - 1, jnp.inf, qsk))
        plsc.store_scatter(qlpad_v, [addr], qsm)
        plsc.store_scatter(ppad_v, [addr], pm)
        plsc.store_scatter(spad_v, [addr], slope)

    iota16 = lax.iota(jnp.int32, 16)
    d0 = iota16
    d1 = (iota16 + 16) % _D
    sv0 = plsc.load_gather(s_v, [d0])
    sv1 = plsc.load_gather(s_v, [d1])
    halves = []
    for col, dd, sv in ((0, d0, sv0), (16, d1, sv1)):
        qb = dd * _STRIDE
        q63 = plsc.load_gather(qpad_v, [qb + 63])
        q31 = plsc.load_gather(qpad_v, [qb + 31])
        q95 = plsc.load_gather(qpad_v, [qb + 95])
        halves.append((col, qb, 1.0 / sv, q63, q31, q95))
    m10 = iota16 < (_D - 16)

    xcopy.wait()

    for chunk in range(_CHUNKS):
        if chunk > 0:
            pltpu.sync_copy(
                x_hbm.at[pl.ds(rbase + chunk * _CROWS, _CROWS), :], x_v)

        @plsc.parallel_loop(0, _CROWS, step=1, unroll=4)
        def body(r):
            rr = jnp.broadcast_to(r, (16,))
            for h, (col, qb, iv, q63, q31, q95) in enumerate(halves):
                if h == 0:
                    xv = x_v[r, pl.ds(0, 16)]
                else:
                    xv = plsc.load_gather(x_v, [rr, d1])

                apos = jnp.where(q63 < xv, qb + 64, qb)
                apos = jnp.where(jnp.where(q63 < xv, q95, q31) < xv,
                                 apos + 32, apos)
                for step in (16, 8, 4, 2, 1):
                    qv = plsc.load_gather(qpad_v, [apos + (step - 1)])
                    apos = jnp.where(qv < xv, apos + step, apos)
                idx = apos - qb

                qls = plsc.load_gather(qlpad_v, [apos])
                pL = plsc.load_gather(ppad_v, [apos])
                sl = plsc.load_gather(spad_v, [apos])

                res = pL + (xv - qls) * sl
                mlow = (idx == 0) & (xv < qls)
                mhigh = (idx == _K) & (xv > qls)
                ta = jnp.abs(xv - qls) * iv
                e = jnp.exp(-2.0 * ta)
                th = (1.0 - e) / (1.0 + e)
                res = jnp.where(mlow, pL - pL * th, res)
                res = jnp.where(mhigh, pL + (1.0 - pL) * th, res)
                if h == 0:
                    o_v[r, pl.ds(0, 16)] = res
                else:
                    plsc.store_scatter(o_v, [rr, d1], res, mask=m10)

        pltpu.sync_copy(
            o_v, out_hbm.at[pl.ds(rbase + chunk * _CROWS, _CROWS), :])


@jax.jit
def _qnorm(xp, quantiles, probs, initial_means, initial_stds):
    mesh = plsc.VectorSubcoreMesh(core_axis_name="c", subcore_axis_name="s")
    f = pl.kernel(
        _body,
        out_type=jax.ShapeDtypeStruct((_B, _D), jnp.float32),
        mesh=mesh,
        compiler_params=pltpu.CompilerParams(
            needs_layout_passes=False, use_tc_tiling_on_sc=True),
        scratch_types=[
            pltpu.VMEM((_CROWS, _D), jnp.float32),
            pltpu.VMEM((_CROWS, _D), jnp.float32),
            pltpu.VMEM((_D, _K), jnp.float32),
            pltpu.VMEM((_K,), jnp.float32),
            pltpu.VMEM((_D,), jnp.float32),
            pltpu.VMEM((_D,), jnp.float32),
            pltpu.VMEM((_D * _STRIDE,), jnp.float32),
            pltpu.VMEM((_D * _STRIDE,), jnp.float32),
            pltpu.VMEM((_D * _STRIDE,), jnp.float32),
            pltpu.VMEM((_D * _STRIDE,), jnp.float32),
            pltpu.SemaphoreType.DMA,
        ],
    )
    return f(xp, quantiles, probs, initial_means, initial_stds)


def kernel(x, quantiles, probs, initial_means, initial_stds):
    return _qnorm(x, quantiles, probs, initial_means, initial_stds)
